# Initial kernel scaffold; baseline (speedup 1.0000x reference)
import jax
import jax.numpy as jnp
from jax.experimental import pallas as pl


def kernel(stims, embed_table, name_map, atn_tensor, W1, b1, W2, b2):
    raise NotImplementedError("write your pallas kernel here")


# speedup vs baseline: 13.7755x; 13.7755x over previous
import functools

import jax
import jax.numpy as jnp
from jax import lax
from jax.experimental import pallas as pl
from jax.experimental.pallas import tpu as pltpu
from jax.experimental.pallas import tpu_sc as plsc

_H = 256
_V = 100000
_B = 2048
_K = 128
_NC = 2
_NS = 16
_NW = _NC * _NS
_APW = _B // _NW
_VBLK = 2000
_L = 16


def _tc_transform_body(emb_ref, w2_ref, b2_ref, stims_ref, w1_ref, b1_ref,
                       t_ref, x_ref):
    t_ref[...] = jnp.maximum(
        jnp.dot(emb_ref[...], w2_ref[...],
                preferred_element_type=jnp.float32) + b2_ref[...], 0.0)

    @pl.when(pl.program_id(0) == 0)
    def _():
        x_ref[...] = jnp.maximum(
            jnp.dot(stims_ref[...], w1_ref[...],
                    preferred_element_type=jnp.float32) + b1_ref[...], 0.0)


def _tc_transform(embed_table, W2, b2, stims, W1, b1, *, interpret=False):
    return pl.pallas_call(
        _tc_transform_body,
        grid=(_V // _VBLK,),
        in_specs=[
            pl.BlockSpec((_VBLK, _H), lambda i: (i, 0)),
            pl.BlockSpec((_H, _H), lambda i: (0, 0)),
            pl.BlockSpec((1, _H), lambda i: (0, 0)),
            pl.BlockSpec((_B, _H), lambda i: (0, 0)),
            pl.BlockSpec((_H, _H), lambda i: (0, 0)),
            pl.BlockSpec((1, _H), lambda i: (0, 0)),
        ],
        out_specs=[
            pl.BlockSpec((_VBLK, _H), lambda i: (i, 0)),
            pl.BlockSpec((_B, _H), lambda i: (0, 0)),
        ],
        out_shape=[
            jax.ShapeDtypeStruct((_V, _H), jnp.float32),
            jax.ShapeDtypeStruct((_B, _H), jnp.float32),
        ],
        interpret=interpret,
    )(embed_table, W2, b2, stims, W1, b1)


def _sc_body(t_hbm, x_hbm, nm_hbm, atn_hbm, scores_hbm, idx_hbm,
             atn_v, names_v, x_v, rows_v, sc_v, ix_v, sem_n, sem_r):
    wid = lax.axis_index("s") * _NC + lax.axis_index("c")
    base = wid * _APW
    pltpu.sync_copy(atn_hbm.at[pl.ds(base, _APW)], atn_v)
    pltpu.sync_copy(x_hbm.at[pl.ds(base, _APW)], x_v)
    lanes = lax.iota(jnp.int32, _L)

    def agent_body(a, carry):
        pltpu.async_copy(nm_hbm.at[atn_v.at[a]], names_v.at[a], sem_n).wait()
        pltpu.async_copy(t_hbm.at[names_v.at[a]], rows_v, sem_r).wait()
        xc = [x_v[a, pl.ds(c * _L, _L)] for c in range(_H // _L)]

        def kg_body(kg, kcarry):
            bval, bidx = kcarry
            sv = jnp.zeros((_L,), jnp.float32)
            for kk in range(_L):
                k = kg * _L + kk
                acc = xc[0] * rows_v[k, pl.ds(0, _L)]
                for c in range(1, _H // _L):
                    acc = acc + xc[c] * rows_v[k, pl.ds(c * _L, _L)]
                s = jnp.sum(acc)
                sv = jnp.where(lanes == kk, s, sv)
            sc_v[a, pl.ds(kg * _L, _L)] = sv
            upd = sv > bval
            bval = jnp.where(upd, sv, bval)
            bidx = jnp.where(upd, kg * _L + lanes, bidx)
            return bval, bidx

        bval0 = jnp.full((_L,), -1.0, jnp.float32)
        bidx0 = jnp.zeros((_L,), jnp.int32)
        bval, bidx = lax.fori_loop(0, _K // _L, kg_body, (bval0, bidx0))
        m = jnp.max(bval)
        cand = jnp.where(bval == m, bidx, jnp.int32(_K))
        amax = jnp.min(cand)
        ix_v[a, ...] = jnp.full((_L,), amax, jnp.int32)
        return carry

    lax.fori_loop(0, _APW, agent_body, 0)
    pltpu.sync_copy(sc_v, scores_hbm.at[pl.ds(base, _APW)])
    pltpu.sync_copy(ix_v, idx_hbm.at[pl.ds(base, _APW)])


def _sc_classify(T, X, name_map, atn, *, interpret=False):
    mesh = plsc.VectorSubcoreMesh(core_axis_name="c", subcore_axis_name="s",
                                  num_cores=_NC, num_subcores=_NS)
    fn = pl.kernel(
        _sc_body,
        mesh=mesh,
        out_type=(
            jax.ShapeDtypeStruct((_B, _K), jnp.float32),
            jax.ShapeDtypeStruct((_B, _L), jnp.int32),
        ),
        scratch_types=[
            pltpu.VMEM((_APW, _K), jnp.int32),
            pltpu.VMEM((_APW, _K), jnp.int32),
            pltpu.VMEM((_APW, _H), jnp.float32),
            pltpu.VMEM((_K, _H), jnp.float32),
            pltpu.VMEM((_APW, _K), jnp.float32),
            pltpu.VMEM((_APW, _L), jnp.int32),
            pltpu.SemaphoreType.DMA,
            pltpu.SemaphoreType.DMA,
        ],
        compiler_params=pltpu.CompilerParams(needs_layout_passes=False),
        interpret=interpret,
    )
    return fn(T, X, name_map, atn)


def kernel(stims, embed_table, name_map, atn_tensor, W1, b1, W2, b2):
    i, j, k, _n = atn_tensor.shape
    T, X = _tc_transform(embed_table, W2, b2.reshape(1, _H), stims, W1,
                         b1.reshape(1, _H))
    atn = atn_tensor.reshape(_B, _K)
    scores, idx16 = _sc_classify(T, X, name_map, atn)
    return scores.reshape(i, j, k), idx16[:, :1].reshape(i, j)


# speedup vs baseline: 19.1501x; 1.3902x over previous
import functools

import jax
import jax.numpy as jnp
from jax import lax
from jax.experimental import pallas as pl
from jax.experimental.pallas import tpu as pltpu
from jax.experimental.pallas import tpu_sc as plsc

_H = 256
_V = 100000
_B = 2048
_K = 128
_NC = 2
_NS = 16
_NW = _NC * _NS
_APW = _B // _NW
_VBLK = 2000
_L = 16


def _tc_transform_body(emb_ref, w2_ref, b2_ref, stims_ref, w1_ref, b1_ref,
                       t_ref, x_ref):
    t_ref[...] = jnp.maximum(
        jnp.dot(emb_ref[...], w2_ref[...],
                preferred_element_type=jnp.float32) + b2_ref[...], 0.0)

    @pl.when(pl.program_id(0) == 0)
    def _():
        x_ref[...] = jnp.maximum(
            jnp.dot(stims_ref[...], w1_ref[...],
                    preferred_element_type=jnp.float32) + b1_ref[...], 0.0)


def _tc_transform(embed_table, W2, b2, stims, W1, b1, *, interpret=False):
    return pl.pallas_call(
        _tc_transform_body,
        grid=(_V // _VBLK,),
        in_specs=[
            pl.BlockSpec((_VBLK, _H), lambda i: (i, 0)),
            pl.BlockSpec((_H, _H), lambda i: (0, 0)),
            pl.BlockSpec((1, _H), lambda i: (0, 0)),
            pl.BlockSpec((_B, _H), lambda i: (0, 0)),
            pl.BlockSpec((_H, _H), lambda i: (0, 0)),
            pl.BlockSpec((1, _H), lambda i: (0, 0)),
        ],
        out_specs=[
            pl.BlockSpec((_VBLK, _H), lambda i: (i, 0)),
            pl.BlockSpec((_B, _H), lambda i: (0, 0)),
        ],
        out_shape=[
            jax.ShapeDtypeStruct((_V, _H), jnp.float32),
            jax.ShapeDtypeStruct((_B, _H), jnp.float32),
        ],
        interpret=interpret,
    )(embed_table, W2, b2, stims, W1, b1)


def _sc_body(t_hbm, x_hbm, nm_hbm, atn_hbm, scores_hbm, idx_hbm,
             atn_v, names_v, x_v, rows_v, sc_v, ix_v, sem_n, sem_r0, sem_r1):
    wid = lax.axis_index("s") * _NC + lax.axis_index("c")
    base = wid * _APW
    pltpu.sync_copy(atn_hbm.at[pl.ds(base, _APW)], atn_v)
    pltpu.sync_copy(x_hbm.at[pl.ds(base, _APW)], x_v)
    lanes = lax.iota(jnp.int32, _L)

    def name_chunk(ch, carry):
        for u in range(8):
            a = ch * 8 + u
            pltpu.async_copy(nm_hbm.at[atn_v.at[a]], names_v.at[a], sem_n)
        for u in range(8):
            a = ch * 8 + u
            pltpu.make_async_copy(nm_hbm.at[atn_v.at[a]], names_v.at[a],
                                  sem_n).wait()
        return carry

    lax.fori_loop(0, _APW // 8, name_chunk, 0)

    def compute(a, slot):
        xc = [x_v[a, pl.ds(c * _L, _L)] for c in range(_H // _L)]

        def kg_body(kg, kcarry):
            bval, bidx = kcarry
            sv = jnp.zeros((_L,), jnp.float32)
            for kk in range(_L):
                k = kg * _L + kk
                acc = xc[0] * rows_v[slot, k, pl.ds(0, _L)]
                for c in range(1, _H // _L):
                    acc = acc + xc[c] * rows_v[slot, k, pl.ds(c * _L, _L)]
                s = jnp.sum(acc)
                sv = jnp.where(lanes == kk, s, sv)
            sc_v[a, pl.ds(kg * _L, _L)] = sv
            upd = sv > bval
            bval = jnp.where(upd, sv, bval)
            bidx = jnp.where(upd, kg * _L + lanes, bidx)
            return bval, bidx

        bval0 = jnp.full((_L,), -1.0, jnp.float32)
        bidx0 = jnp.zeros((_L,), jnp.int32)
        bval, bidx = lax.fori_loop(0, _K // _L, kg_body, (bval0, bidx0))
        m = jnp.max(bval)
        cand = jnp.where(bval == m, bidx, jnp.int32(_K))
        amax = jnp.min(cand)
        ix_v[a, ...] = jnp.full((_L,), amax, jnp.int32)

    def row_gather(a, slot, sem):
        return pltpu.async_copy(t_hbm.at[names_v.at[a]], rows_v.at[slot], sem)

    def row_wait(a, slot, sem):
        pltpu.make_async_copy(t_hbm.at[names_v.at[a]], rows_v.at[slot],
                              sem).wait()

    row_gather(0, 0, sem_r0)
    row_gather(1, 1, sem_r1)

    def pair_body(p, carry):
        a0 = 2 * p
        a1 = a0 + 1
        row_wait(a0, 0, sem_r0)
        compute(a0, 0)

        @pl.when(p + 1 < _APW // 2)
        def _():
            row_gather(a0 + 2, 0, sem_r0)

        row_wait(a1, 1, sem_r1)
        compute(a1, 1)

        @pl.when(p + 1 < _APW // 2)
        def _():
            row_gather(a1 + 2, 1, sem_r1)

        return carry

    lax.fori_loop(0, _APW // 2, pair_body, 0)
    pltpu.sync_copy(sc_v, scores_hbm.at[pl.ds(base, _APW)])
    pltpu.sync_copy(ix_v, idx_hbm.at[pl.ds(base, _APW)])


def _sc_classify(T, X, name_map, atn, *, interpret=False):
    mesh = plsc.VectorSubcoreMesh(core_axis_name="c", subcore_axis_name="s",
                                  num_cores=_NC, num_subcores=_NS)
    fn = pl.kernel(
        _sc_body,
        mesh=mesh,
        out_type=(
            jax.ShapeDtypeStruct((_B, _K), jnp.float32),
            jax.ShapeDtypeStruct((_B, _L), jnp.int32),
        ),
        scratch_types=[
            pltpu.VMEM((_APW, _K), jnp.int32),
            pltpu.VMEM((_APW, _K), jnp.int32),
            pltpu.VMEM((_APW, _H), jnp.float32),
            pltpu.VMEM((2, _K, _H), jnp.float32),
            pltpu.VMEM((_APW, _K), jnp.float32),
            pltpu.VMEM((_APW, _L), jnp.int32),
            pltpu.SemaphoreType.DMA,
            pltpu.SemaphoreType.DMA,
            pltpu.SemaphoreType.DMA,
        ],
        compiler_params=pltpu.CompilerParams(needs_layout_passes=False),
        interpret=interpret,
    )
    return fn(T, X, name_map, atn)


def kernel(stims, embed_table, name_map, atn_tensor, W1, b1, W2, b2):
    i, j, k, _n = atn_tensor.shape
    T, X = _tc_transform(embed_table, W2, b2.reshape(1, _H), stims, W1,
                         b1.reshape(1, _H))
    atn = atn_tensor.reshape(_B, _K)
    scores, idx16 = _sc_classify(T, X, name_map, atn)
    return scores.reshape(i, j, k), idx16[:, :1].reshape(i, j)
